# baseline (speedup 1.0000x reference)
import functools

import jax
import jax.numpy as jnp
from jax import lax
from jax.experimental import pallas as pl
from jax.experimental.pallas import tpu as pltpu
from jax.experimental.pallas import tpu_sc as plsc

D_MODEL = 64
SCALE = float(D_MODEL) ** 0.5
NC, NS, L = 2, 16, 16
NW = NC * NS
B = 4096 * 200
BPW = B // NW
R = 128
G = 8
C = R * G
NCHUNK = BPW // C


def _body(x_hbm, table_hbm, out_hbm, ibuf, rows, gsem):
    wid = lax.axis_index("s") * NC + lax.axis_index("c")

    def chunk(g, carry):
        irow = wid * (BPW // R) + g * G
        pltpu.sync_copy(x_hbm.at[pl.ds(irow, G)], ibuf)
        cps = [
            pltpu.async_copy(
                table_hbm.at[ibuf.at[j]], rows.at[pl.ds(j * R, R)], gsem
            )
            for j in range(G)
        ]
        for cp in cps:
            cp.wait()

        def srow(i, c2):
            for j in range(D_MODEL // L):
                rows[i, pl.ds(j * L, L)] = rows[i, pl.ds(j * L, L)] * SCALE
            return c2

        lax.fori_loop(0, C, srow, 0)
        pltpu.sync_copy(rows, out_hbm.at[pl.ds(wid * BPW + g * C, C)])
        return carry

    lax.fori_loop(0, NCHUNK, chunk, 0)


@jax.jit
def _run(x2, table):
    mesh = plsc.VectorSubcoreMesh(core_axis_name="c", subcore_axis_name="s")
    f = functools.partial(
        pl.kernel,
        out_type=jax.ShapeDtypeStruct((B, D_MODEL), jnp.float32),
        mesh=mesh,
        compiler_params=pltpu.CompilerParams(use_tc_tiling_on_sc=False),
        scratch_types=[
            pltpu.VMEM((G, R), jnp.int32),
            pltpu.VMEM((C, D_MODEL), jnp.float32),
            pltpu.SemaphoreType.DMA,
        ],
    )(_body)
    return f(x2, table)


def kernel(x, table):
    x2 = x.reshape(B // R, R)
    out = _run(x2, table)
    return out.reshape(x.shape + (D_MODEL,))


# speedup vs baseline: 1.1024x; 1.1024x over previous
import functools

import jax
import jax.numpy as jnp
from jax import lax
from jax.experimental import pallas as pl
from jax.experimental.pallas import tpu as pltpu
from jax.experimental.pallas import tpu_sc as plsc

D_MODEL = 64
SCALE = float(D_MODEL) ** 0.5
NC, NS, L = 2, 16, 16
NW = NC * NS
B = 4096 * 200
BPW = B // NW
R = 128
IR = BPW // R
GPC = 5
CK = R * GPC
NCHK = BPW // CK
NPAIR = NCHK // 2


def _body(x_hbm, table_hbm, out_hbm,
          ibuf, rows0, rows1, gsem0, gsem1, ssem0, ssem1):
    wid = lax.axis_index("s") * NC + lax.axis_index("c")
    out_base = wid * BPW

    pltpu.sync_copy(x_hbm.at[pl.ds(wid * IR, IR)], ibuf)

    def gathers(k, rbuf, sem):
        return [
            pltpu.async_copy(
                table_hbm.at[ibuf.at[k * GPC + j]],
                rbuf.at[pl.ds(j * R, R)], sem,
            )
            for j in range(GPC)
        ]

    def scale(rbuf):
        @plsc.parallel_loop(0, CK, unroll=8)
        def _(i):
            for j in range(D_MODEL // L):
                rbuf[i, pl.ds(j * L, L)] = rbuf[i, pl.ds(j * L, L)] * SCALE

    def store(k, rbuf, sem):
        pltpu.async_copy(rbuf, out_hbm.at[pl.ds(out_base + k * CK, CK)], sem)

    def store_wait(rbuf, sem):
        pltpu.make_async_copy(rbuf, out_hbm.at[pl.ds(out_base, CK)], sem).wait()

    def pair(t, carry):
        k0 = 2 * t
        k1 = 2 * t + 1

        @pl.when(t > 0)
        def _():
            store_wait(rows0, ssem0)

        g0 = gathers(k0, rows0, gsem0)

        @pl.when(t > 0)
        def _():
            store_wait(rows1, ssem1)

        g1 = gathers(k1, rows1, gsem1)

        for cp in g0:
            cp.wait()
        scale(rows0)
        store(k0, rows0, ssem0)

        for cp in g1:
            cp.wait()
        scale(rows1)
        store(k1, rows1, ssem1)
        return carry

    lax.fori_loop(0, NPAIR, pair, 0)
    store_wait(rows0, ssem0)
    store_wait(rows1, ssem1)


@jax.jit
def _run(x2, table):
    mesh = plsc.VectorSubcoreMesh(core_axis_name="c", subcore_axis_name="s")
    f = functools.partial(
        pl.kernel,
        out_type=jax.ShapeDtypeStruct((B, D_MODEL), jnp.float32),
        mesh=mesh,
        compiler_params=pltpu.CompilerParams(use_tc_tiling_on_sc=False),
        scratch_types=[
            pltpu.VMEM((IR, R), jnp.int32),
            pltpu.VMEM((CK, D_MODEL), jnp.float32),
            pltpu.VMEM((CK, D_MODEL), jnp.float32),
            pltpu.SemaphoreType.DMA,
            pltpu.SemaphoreType.DMA,
            pltpu.SemaphoreType.DMA,
            pltpu.SemaphoreType.DMA,
        ],
    )(_body)
    return f(x2, table)


def kernel(x, table):
    x2 = x.reshape(B // R, R)
    out = _run(x2, table)
    return out.reshape(x.shape + (D_MODEL,))


# speedup vs baseline: 1.1028x; 1.0004x over previous
import functools

import jax
import jax.numpy as jnp
from jax import lax
from jax.experimental import pallas as pl
from jax.experimental.pallas import tpu as pltpu
from jax.experimental.pallas import tpu_sc as plsc

BATCH = 4096
SEQ = 200
D_MODEL = 64
SCALE = float(D_MODEL) ** 0.5
NC, NS, L = 2, 16, 16
NW = NC * NS
SPW = BATCH // NW
SPC = 4
CK = SPC * SEQ
SPLITS = ((0, 104), (104, 96))
NCHK = SPW // SPC
NPAIR = NCHK // 2


def _body(x_hbm, table_hbm, out_hbm,
          ibuf, rows0, rows1, gsem0, gsem1, ssem0, ssem1):
    wid = lax.axis_index("s") * NC + lax.axis_index("c")
    seq_base = wid * SPW

    pltpu.sync_copy(x_hbm.at[pl.ds(seq_base, SPW)], ibuf)

    def gathers(k, rbuf, sem):
        cps = []
        for s in range(SPC):
            for off, n in SPLITS:
                cps.append(pltpu.async_copy(
                    table_hbm.at[ibuf.at[k * SPC + s, pl.ds(off, n)]],
                    rbuf.at[s, pl.ds(off, n)], sem,
                ))
        return cps

    def scale(rbuf):
        @plsc.parallel_loop(0, SPC * SEQ, unroll=8)
        def _(i):
            s = i // SEQ
            t = i % SEQ
            for j in range(D_MODEL // L):
                rbuf[s, t, pl.ds(j * L, L)] = rbuf[s, t, pl.ds(j * L, L)] * SCALE

    def store(k, rbuf, sem):
        pltpu.async_copy(rbuf, out_hbm.at[pl.ds(seq_base + k * SPC, SPC)], sem)

    def store_wait(rbuf, sem):
        pltpu.make_async_copy(rbuf, out_hbm.at[pl.ds(seq_base, SPC)], sem).wait()

    def pair(t, carry):
        k0 = 2 * t
        k1 = 2 * t + 1

        @pl.when(t > 0)
        def _():
            store_wait(rows0, ssem0)

        g0 = gathers(k0, rows0, gsem0)

        @pl.when(t > 0)
        def _():
            store_wait(rows1, ssem1)

        g1 = gathers(k1, rows1, gsem1)

        for cp in g0:
            cp.wait()
        scale(rows0)
        store(k0, rows0, ssem0)

        for cp in g1:
            cp.wait()
        scale(rows1)
        store(k1, rows1, ssem1)
        return carry

    lax.fori_loop(0, NPAIR, pair, 0)
    store_wait(rows0, ssem0)
    store_wait(rows1, ssem1)


@jax.jit
def _run(x, table):
    mesh = plsc.VectorSubcoreMesh(core_axis_name="c", subcore_axis_name="s")
    f = functools.partial(
        pl.kernel,
        out_type=jax.ShapeDtypeStruct((BATCH, SEQ, D_MODEL), jnp.float32),
        mesh=mesh,
        compiler_params=pltpu.CompilerParams(use_tc_tiling_on_sc=False),
        scratch_types=[
            pltpu.VMEM((SPW, SEQ), jnp.int32),
            pltpu.VMEM((SPC, SEQ, D_MODEL), jnp.float32),
            pltpu.VMEM((SPC, SEQ, D_MODEL), jnp.float32),
            pltpu.SemaphoreType.DMA,
            pltpu.SemaphoreType.DMA,
            pltpu.SemaphoreType.DMA,
            pltpu.SemaphoreType.DMA,
        ],
    )(_body)
    return f(x, table)


def kernel(x, table):
    return _run(x, table)
